# baseline (speedup 1.0000x reference)
import functools

import jax
import jax.numpy as jnp
from jax import lax
from jax.experimental import pallas as pl
from jax.experimental.pallas import tpu as pltpu
from jax.experimental.pallas import tpu_sc as plsc

_N = 10000
_E = 320000
_DEG = _E // _N
_DL = 32
_LANES = 16
_W = 32
_C = 16
_ECH = _C * _DEG
_GSUB = 512
_NPW = 320
_NCHUNK = _NPW // _C

_PERM = list(range(0, _DL, 2)) + list(range(1, _DL, 2))


def _sigmoid(x):
    return 1.0 / (1.0 + jnp.exp(-x))


def _eluplus(x):
    return jnp.where(x > 0, x + 1.0, jnp.exp(x))


def _tables_body(x_ref, wql_ref, wkl_ref, qk_ref, xsum_ref):
    xb = x_ref[...]
    inv_d = 1.0 / 128.0
    wq = _sigmoid(wql_ref[...])
    wq = wq / jnp.sum(wq)
    qk_ref[:, 0:_DL] = (jnp.dot(xb, wq.T, preferred_element_type=jnp.float32)
                        * inv_d).astype(jnp.bfloat16)
    qk_ref[:, _DL:2 * _DL] = (jnp.dot(xb, _eluplus(wkl_ref[...]).T,
                                      preferred_element_type=jnp.float32)
                              * inv_d).astype(jnp.bfloat16)

    @pl.when(pl.program_id(0) == 0)
    def _():
        xsum_ref[...] = jnp.zeros_like(xsum_ref)

    xsum_ref[...] += jnp.sum(xb, axis=0, keepdims=True)


def _tables(x, w_q_local, w_k_local):
    n, d = x.shape
    blk = 2000
    full = lambda i: (0, 0)
    row = lambda i: (i, 0)
    return pl.pallas_call(
        _tables_body,
        grid=(n // blk,),
        in_specs=[
            pl.BlockSpec((blk, d), row),
            pl.BlockSpec(w_q_local.shape, full),
            pl.BlockSpec(w_k_local.shape, full),
        ],
        out_specs=[
            pl.BlockSpec((blk, 2 * _DL), row),
            pl.BlockSpec((1, d), full),
        ],
        out_shape=[
            jax.ShapeDtypeStruct((n, 2 * _DL), jnp.bfloat16),
            jax.ShapeDtypeStruct((1, d), jnp.float32),
        ],
    )(x, w_q_local, w_k_local)


def _split_bf16_row(row):
    bits = plsc.bitcast(row, jnp.int32)
    even = plsc.bitcast(jnp.left_shift(bits, 16), jnp.float32)
    odd = plsc.bitcast(bits, jnp.float32)
    return even, odd


def _sc_local_body(adj, qk, out_hbm,
                   idxk_all, idxq_all, qrows, krows, outv, qksh, gsem, osem):
    cid = lax.axis_index("c")
    sid = lax.axis_index("s")
    wid = sid * 2 + cid
    rows_per_tile = _N // 16
    seg = pl.ds(sid * rows_per_tile, rows_per_tile)
    pltpu.sync_copy(qk.at[seg], qksh.at[seg])
    plsc.subcore_barrier()
    nodes_w = jnp.where(wid < 2, _NPW, 312)
    base = wid * 312 + 8 * jnp.minimum(wid, 2)
    last_start = base + nodes_w - _C
    astart = jnp.minimum(base, _N - _NPW)
    pltpu.sync_copy(adj.at[0, pl.ds(astart * _DEG, _NPW * _DEG)], idxk_all)
    pltpu.sync_copy(adj.at[1, pl.ds(astart * _DEG, _NPW * _DEG)], idxq_all)

    def fire(jj, p):
        cs = pl.multiple_of(jnp.minimum(base + jj * _C, last_start), 8)
        boff = (cs - astart) * _DEG
        for k in range(_ECH // _GSUB):
            src = pl.ds(boff + k * _GSUB, _GSUB)
            dst = pl.ds(k * _GSUB, _GSUB)
            pltpu.async_copy(qksh.at[idxq_all.at[src]], qrows.at[p, dst], gsem.at[p])
            pltpu.async_copy(qksh.at[idxk_all.at[src]], krows.at[p, dst], gsem.at[p])

    def drain(p):
        for _ in range(2 * (_ECH // _GSUB)):
            pltpu.make_async_copy(
                qk.at[pl.ds(0, _GSUB)], qrows.at[p, pl.ds(0, _GSUB)],
                gsem.at[p]).wait()

    def compute(jj, p):
        cs = pl.multiple_of(jnp.minimum(base + jj * _C, last_start), 8)

        def node(i, _):
            acc = [jnp.zeros((_LANES,), jnp.float32) for _ in range(4)]
            for e0 in range(_DEG):
                e = i * _DEG + e0
                prod = (qrows[p, e, pl.ds(0, _DL)]
                        * krows[p, e, pl.ds(_DL, _DL)])
                p_ev, p_od = _split_bf16_row(prod)
                c = 2 * (e0 & 1)
                acc[c] = acc[c] + p_ev
                acc[c + 1] = acc[c + 1] + p_od
            outv[p, i, pl.ds(0, _LANES)] = acc[0] + acc[2]
            outv[p, i, pl.ds(_LANES, _LANES)] = acc[1] + acc[3]
            return 0

        lax.fori_loop(0, _C, node, 0)
        pltpu.async_copy(outv.at[p], out_hbm.at[pl.ds(cs, _C)], osem.at[p])

    def wait_out(p):
        pltpu.make_async_copy(outv.at[p], out_hbm.at[pl.ds(0, _C)], osem.at[p]).wait()

    fire(0, 0)

    def body(j, carry):
        for b in range(2):
            jj = j + b
            p = b & 1

            @pl.when(jj < _NCHUNK - 1)
            def _():
                fire(jj + 1, 1 - p)

            drain(p)

            @pl.when(jj >= 2)
            def _():
                wait_out(p)

            compute(jj, p)
        return carry

    lax.fori_loop(0, _NCHUNK // 2, lambda j, c: body(2 * j, c), 0)
    wait_out(0)
    wait_out(1)


def _sc_local(adj, qk):
    mesh = plsc.VectorSubcoreMesh(core_axis_name="c", subcore_axis_name="s")
    run = functools.partial(
        pl.kernel,
        out_type=jax.ShapeDtypeStruct((_N, _DL), jnp.float32),
        mesh=mesh,
        scratch_types=[
            pltpu.VMEM((_NPW * _DEG,), jnp.int32),
            pltpu.VMEM((_NPW * _DEG,), jnp.int32),
            pltpu.VMEM((2, _ECH, 2 * _DL), jnp.bfloat16),
            pltpu.VMEM((2, _ECH, 2 * _DL), jnp.bfloat16),
            pltpu.VMEM((2, _C, _DL), jnp.float32),
            pltpu.VMEM_SHARED((_N, 2 * _DL), jnp.bfloat16),
            pltpu.SemaphoreType.DMA((2,)),
            pltpu.SemaphoreType.DMA((2,)),
        ],
        compiler_params=pltpu.CompilerParams(use_tc_tiling_on_sc=False,
                                             needs_layout_passes=False),
    )(_sc_local_body)
    return run(adj, qk)


def _egoglobal_body(x_ref, wqe_ref, wqg_ref, wkg_ref, xsum_ref,
                    wve_ref, wvg_ref, peg_ref, se_ref):
    xb = x_ref[...]
    inv_d = 1.0 / 128.0
    emb = jnp.dot(xb, wqe_ref[...].T, preferred_element_type=jnp.float32) * inv_d
    ego = emb * emb
    wqg = _sigmoid(wqg_ref[...])
    wqg = wqg / jnp.sum(wqg)
    qg = jnp.dot(xb, wqg.T, preferred_element_type=jnp.float32) * inv_d
    xbar = xsum_ref[...] * (1.0 / _N)
    kg = jnp.dot(xbar, _eluplus(wkg_ref[...]).T,
                 preferred_element_type=jnp.float32) * inv_d
    gs = qg * kg
    peg_ref[...] = (
        jnp.dot(ego, _eluplus(wve_ref[...]).T, preferred_element_type=jnp.float32)
        + jnp.dot(gs, _eluplus(wvg_ref[...]).T, preferred_element_type=jnp.float32))
    se_ref[...] = (0.001
                   + jnp.sum(ego, axis=1, keepdims=True)
                   + jnp.sum(gs, axis=1, keepdims=True))


def _egoglobal(x, w_qk_ego, w_q_global, w_k_global, xsum, w_v_ego, w_v_global):
    n, d = x.shape
    blk = 2000
    full = lambda i: (0, 0)
    row = lambda i: (i, 0)
    return pl.pallas_call(
        _egoglobal_body,
        grid=(n // blk,),
        in_specs=[
            pl.BlockSpec((blk, d), row),
            pl.BlockSpec(w_qk_ego.shape, full),
            pl.BlockSpec(w_q_global.shape, full),
            pl.BlockSpec(w_k_global.shape, full),
            pl.BlockSpec((1, d), full),
            pl.BlockSpec(w_v_ego.shape, full),
            pl.BlockSpec(w_v_global.shape, full),
        ],
        out_specs=[
            pl.BlockSpec((blk, 128), row),
            pl.BlockSpec((blk, 1), row),
        ],
        out_shape=[
            jax.ShapeDtypeStruct((n, 128), jnp.float32),
            jax.ShapeDtypeStruct((n, 1), jnp.float32),
        ],
    )(x, w_qk_ego, w_q_global, w_k_global, xsum, w_v_ego, w_v_global)


def _combine_body(loc_ref, peg_ref, se_ref, wvl_ref, bias_ref, out_ref):
    loc = loc_ref[...]
    s = se_ref[...] + jnp.sum(loc, axis=1, keepdims=True)
    inv = 1.0 / s
    res = peg_ref[...] + jnp.dot(loc, _eluplus(wvl_ref[...]).T,
                                 preferred_element_type=jnp.float32)
    out_ref[...] = res * inv + _eluplus(bias_ref[...])


def _combine(loc, peg, se, w_v_local, bias_p):
    n = loc.shape[0]
    blk = 2000
    full = lambda i: (0, 0)
    row = lambda i: (i, 0)
    return pl.pallas_call(
        _combine_body,
        grid=(n // blk,),
        in_specs=[
            pl.BlockSpec((blk, _DL), row),
            pl.BlockSpec((blk, 128), row),
            pl.BlockSpec((blk, 1), row),
            pl.BlockSpec(w_v_local.shape, full),
            pl.BlockSpec(bias_p.shape, full),
        ],
        out_specs=pl.BlockSpec((blk, 128), row),
        out_shape=jax.ShapeDtypeStruct((n, 128), jnp.float32),
    )(loc, peg, se, w_v_local, bias_p)


def kernel(adj_matrix, x, w_qk_ego, w_v_ego, w_q_local, w_k_local, w_v_local,
           w_q_global, w_k_global, w_v_global, bias_p):
    qk, xsum = _tables(x, w_q_local, w_k_local)
    loc_perm = _sc_local(adj_matrix, qk)
    perm = jnp.asarray(_PERM, dtype=jnp.int32)
    peg, se = _egoglobal(x, w_qk_ego, w_q_global, w_k_global, xsum,
                         w_v_ego, w_v_global)
    return _combine(loc_perm, peg, se, w_v_local[:, perm], bias_p)


# speedup vs baseline: 1.0813x; 1.0813x over previous
import functools

import jax
import jax.numpy as jnp
from jax import lax
from jax.experimental import pallas as pl
from jax.experimental.pallas import tpu as pltpu
from jax.experimental.pallas import tpu_sc as plsc

_N = 10000
_E = 320000
_DEG = _E // _N
_DL = 32
_LANES = 16
_W = 32
_C = 32
_ECH = _C * _DEG
_GSUB = 512
_NPW = 320
_NCHUNK = _NPW // _C

_PERM = list(range(0, _DL, 2)) + list(range(1, _DL, 2))


def _sigmoid(x):
    return 1.0 / (1.0 + jnp.exp(-x))


def _eluplus(x):
    return jnp.where(x > 0, x + 1.0, jnp.exp(x))


def _tables_body(x_ref, wql_ref, wkl_ref, qk_ref, xsum_ref):
    xb = x_ref[...]
    inv_d = 1.0 / 128.0
    wq = _sigmoid(wql_ref[...])
    wq = wq / jnp.sum(wq)
    qk_ref[0] = (jnp.dot(xb, wq.T, preferred_element_type=jnp.float32)
                 * inv_d).astype(jnp.bfloat16)
    qk_ref[1] = (jnp.dot(xb, _eluplus(wkl_ref[...]).T,
                         preferred_element_type=jnp.float32)
                 * inv_d).astype(jnp.bfloat16)

    @pl.when(pl.program_id(0) == 0)
    def _():
        xsum_ref[...] = jnp.zeros_like(xsum_ref)

    xsum_ref[...] += jnp.sum(xb, axis=0, keepdims=True)


def _tables(x, w_q_local, w_k_local):
    n, d = x.shape
    blk = 2000
    full = lambda i: (0, 0)
    row = lambda i: (i, 0)
    return pl.pallas_call(
        _tables_body,
        grid=(n // blk,),
        in_specs=[
            pl.BlockSpec((blk, d), row),
            pl.BlockSpec(w_q_local.shape, full),
            pl.BlockSpec(w_k_local.shape, full),
        ],
        out_specs=[
            pl.BlockSpec((2, blk, _DL), lambda i: (0, i, 0)),
            pl.BlockSpec((1, d), full),
        ],
        out_shape=[
            jax.ShapeDtypeStruct((2, n, _DL), jnp.bfloat16),
            jax.ShapeDtypeStruct((1, d), jnp.float32),
        ],
    )(x, w_q_local, w_k_local)


def _split_bf16_row(row):
    bits = plsc.bitcast(row, jnp.int32)
    even = plsc.bitcast(jnp.left_shift(bits, 16), jnp.float32)
    odd = plsc.bitcast(bits, jnp.float32)
    return even, odd


def _sc_local_body(adj, qk, out_hbm,
                   idxk_all, idxq_all, qrows, krows, outv, qsh, ksh, gsem, osem):
    cid = lax.axis_index("c")
    sid = lax.axis_index("s")
    wid = sid * 2 + cid
    rows_per_tile = _N // 16
    seg = pl.ds(sid * rows_per_tile, rows_per_tile)
    pltpu.sync_copy(qk.at[0, seg], qsh.at[seg])
    pltpu.sync_copy(qk.at[1, seg], ksh.at[seg])
    plsc.subcore_barrier()
    nodes_w = jnp.where(wid < 2, _NPW, 312)
    base = wid * 312 + 8 * jnp.minimum(wid, 2)
    last_start = base + nodes_w - _C
    astart = jnp.minimum(base, _N - _NPW)
    pltpu.sync_copy(adj.at[0, pl.ds(astart * _DEG, _NPW * _DEG)], idxk_all)
    pltpu.sync_copy(adj.at[1, pl.ds(astart * _DEG, _NPW * _DEG)], idxq_all)

    def fire(jj, p):
        cs = pl.multiple_of(jnp.minimum(base + jj * _C, last_start), 8)
        boff = (cs - astart) * _DEG
        for k in range(_ECH // _GSUB):
            src = pl.ds(boff + k * _GSUB, _GSUB)
            dst = pl.ds(k * _GSUB, _GSUB)
            pltpu.async_copy(qsh.at[idxq_all.at[src]], qrows.at[p, dst], gsem.at[p])
            pltpu.async_copy(ksh.at[idxk_all.at[src]], krows.at[p, dst], gsem.at[p])

    def drain(p):
        for _ in range(2 * (_ECH // _GSUB)):
            pltpu.make_async_copy(
                qk.at[0, pl.ds(0, _GSUB)], qrows.at[p, pl.ds(0, _GSUB)],
                gsem.at[p]).wait()

    def compute(jj, p):
        cs = pl.multiple_of(jnp.minimum(base + jj * _C, last_start), 8)

        def node(i, _):
            acc = [jnp.zeros((_LANES,), jnp.float32) for _ in range(4)]
            for e0 in range(_DEG):
                e = i * _DEG + e0
                prod = qrows[p, e, :] * krows[p, e, :]
                p_ev, p_od = _split_bf16_row(prod)
                c = 2 * (e0 & 1)
                acc[c] = acc[c] + p_ev
                acc[c + 1] = acc[c + 1] + p_od
            outv[p, i, pl.ds(0, _LANES)] = acc[0] + acc[2]
            outv[p, i, pl.ds(_LANES, _LANES)] = acc[1] + acc[3]
            return 0

        lax.fori_loop(0, _C, node, 0)
        pltpu.async_copy(outv.at[p], out_hbm.at[pl.ds(cs, _C)], osem.at[p])

    def wait_out(p):
        pltpu.make_async_copy(outv.at[p], out_hbm.at[pl.ds(0, _C)], osem.at[p]).wait()

    fire(0, 0)

    def body(j, carry):
        for b in range(2):
            jj = j + b
            p = b & 1

            @pl.when(jj < _NCHUNK - 1)
            def _():
                fire(jj + 1, 1 - p)

            drain(p)

            @pl.when(jj >= 2)
            def _():
                wait_out(p)

            compute(jj, p)
        return carry

    lax.fori_loop(0, _NCHUNK // 2, lambda j, c: body(2 * j, c), 0)
    wait_out(0)
    wait_out(1)


def _sc_local(adj, qk):
    mesh = plsc.VectorSubcoreMesh(core_axis_name="c", subcore_axis_name="s")
    run = functools.partial(
        pl.kernel,
        out_type=jax.ShapeDtypeStruct((_N, _DL), jnp.float32),
        mesh=mesh,
        scratch_types=[
            pltpu.VMEM((_NPW * _DEG,), jnp.int32),
            pltpu.VMEM((_NPW * _DEG,), jnp.int32),
            pltpu.VMEM((2, _ECH, _DL), jnp.bfloat16),
            pltpu.VMEM((2, _ECH, _DL), jnp.bfloat16),
            pltpu.VMEM((2, _C, _DL), jnp.float32),
            pltpu.VMEM_SHARED((_N, _DL), jnp.bfloat16),
            pltpu.VMEM_SHARED((_N, _DL), jnp.bfloat16),
            pltpu.SemaphoreType.DMA((2,)),
            pltpu.SemaphoreType.DMA((2,)),
        ],
        compiler_params=pltpu.CompilerParams(use_tc_tiling_on_sc=False,
                                             needs_layout_passes=False),
    )(_sc_local_body)
    return run(adj, qk)


def _egoglobal_body(x_ref, wqe_ref, wqg_ref, wkg_ref, xsum_ref,
                    wve_ref, wvg_ref, peg_ref, se_ref):
    xb = x_ref[...]
    inv_d = 1.0 / 128.0
    emb = jnp.dot(xb, wqe_ref[...].T, preferred_element_type=jnp.float32) * inv_d
    ego = emb * emb
    wqg = _sigmoid(wqg_ref[...])
    wqg = wqg / jnp.sum(wqg)
    qg = jnp.dot(xb, wqg.T, preferred_element_type=jnp.float32) * inv_d
    xbar = xsum_ref[...] * (1.0 / _N)
    kg = jnp.dot(xbar, _eluplus(wkg_ref[...]).T,
                 preferred_element_type=jnp.float32) * inv_d
    gs = qg * kg
    peg_ref[...] = (
        jnp.dot(ego, _eluplus(wve_ref[...]).T, preferred_element_type=jnp.float32)
        + jnp.dot(gs, _eluplus(wvg_ref[...]).T, preferred_element_type=jnp.float32))
    se_ref[...] = (0.001
                   + jnp.sum(ego, axis=1, keepdims=True)
                   + jnp.sum(gs, axis=1, keepdims=True))


def _egoglobal(x, w_qk_ego, w_q_global, w_k_global, xsum, w_v_ego, w_v_global):
    n, d = x.shape
    blk = 2000
    full = lambda i: (0, 0)
    row = lambda i: (i, 0)
    return pl.pallas_call(
        _egoglobal_body,
        grid=(n // blk,),
        in_specs=[
            pl.BlockSpec((blk, d), row),
            pl.BlockSpec(w_qk_ego.shape, full),
            pl.BlockSpec(w_q_global.shape, full),
            pl.BlockSpec(w_k_global.shape, full),
            pl.BlockSpec((1, d), full),
            pl.BlockSpec(w_v_ego.shape, full),
            pl.BlockSpec(w_v_global.shape, full),
        ],
        out_specs=[
            pl.BlockSpec((blk, 128), row),
            pl.BlockSpec((blk, 1), row),
        ],
        out_shape=[
            jax.ShapeDtypeStruct((n, 128), jnp.float32),
            jax.ShapeDtypeStruct((n, 1), jnp.float32),
        ],
    )(x, w_qk_ego, w_q_global, w_k_global, xsum, w_v_ego, w_v_global)


def _combine_body(loc_ref, peg_ref, se_ref, wvl_ref, bias_ref, out_ref):
    loc = loc_ref[...]
    s = se_ref[...] + jnp.sum(loc, axis=1, keepdims=True)
    inv = 1.0 / s
    res = peg_ref[...] + jnp.dot(loc, _eluplus(wvl_ref[...]).T,
                                 preferred_element_type=jnp.float32)
    out_ref[...] = res * inv + _eluplus(bias_ref[...])


def _combine(loc, peg, se, w_v_local, bias_p):
    n = loc.shape[0]
    blk = 2000
    full = lambda i: (0, 0)
    row = lambda i: (i, 0)
    return pl.pallas_call(
        _combine_body,
        grid=(n // blk,),
        in_specs=[
            pl.BlockSpec((blk, _DL), row),
            pl.BlockSpec((blk, 128), row),
            pl.BlockSpec((blk, 1), row),
            pl.BlockSpec(w_v_local.shape, full),
            pl.BlockSpec(bias_p.shape, full),
        ],
        out_specs=pl.BlockSpec((blk, 128), row),
        out_shape=jax.ShapeDtypeStruct((n, 128), jnp.float32),
    )(loc, peg, se, w_v_local, bias_p)


def kernel(adj_matrix, x, w_qk_ego, w_v_ego, w_q_local, w_k_local, w_v_local,
           w_q_global, w_k_global, w_v_global, bias_p):
    qk, xsum = _tables(x, w_q_local, w_k_local)
    loc_perm = _sc_local(adj_matrix, qk)
    perm = jnp.asarray(_PERM, dtype=jnp.int32)
    peg, se = _egoglobal(x, w_qk_ego, w_q_global, w_k_global, xsum,
                         w_v_ego, w_v_global)
    return _combine(loc_perm, peg, se, w_v_local[:, perm], bias_p)
